# baseline (speedup 1.0000x reference)
import functools

import jax
import jax.numpy as jnp
from jax import lax
from jax.experimental import pallas as pl
from jax.experimental.pallas import tpu as pltpu
from jax.experimental.pallas import tpu_sc as plsc

_B = 4096
_CTX = 20
_D = 64
_OUT = 100000

_NC = 2
_NS = 16
_NW = _NC * _NS
_BPW = _B // _NW
_SUB = 32
_NSUB = _BPW // _SUB

_LANES = 16


def _sc_pool_body(idx_hbm, table_hbm, pooled_hbm, idx_v, bufs, out_v, sem):
    wid = lax.axis_index("s") * _NC + lax.axis_index("c")
    base = wid * _BPW
    pltpu.sync_copy(idx_hbm.at[pl.ds(base, _BPW), :], idx_v)

    for sc in range(_NSUB):
        descs = [
            pltpu.async_copy(
                table_hbm.at[idx_v.at[sc * _SUB + j]], bufs.at[j], sem
            )
            for j in range(_SUB)
        ]
        for d in descs:
            d.wait()

        def reduce_one(j, carry, sc=sc):
            row = sc * _SUB + j
            for d in range(_D // _LANES):
                sl = pl.ds(d * _LANES, _LANES)
                acc = bufs[j, 0, sl]
                for c in range(1, _CTX):
                    acc = acc + bufs[j, c, sl]
                out_v[row, sl] = acc
            return carry

        lax.fori_loop(0, _SUB, reduce_one, 0)

    pltpu.sync_copy(out_v, pooled_hbm.at[pl.ds(base, _BPW), :])


def _sc_pool(inputs, embed_table):
    mesh = plsc.VectorSubcoreMesh(core_axis_name="c", subcore_axis_name="s")
    return pl.kernel(
        _sc_pool_body,
        out_type=jax.ShapeDtypeStruct((_B, _D), jnp.float32),
        mesh=mesh,
        scratch_types=[
            pltpu.VMEM((_BPW, _CTX), jnp.int32),
            pltpu.VMEM((_SUB, _CTX, _D), jnp.float32),
            pltpu.VMEM((_BPW, _D), jnp.float32),
            pltpu.SemaphoreType.DMA,
        ],
        compiler_params=pltpu.CompilerParams(use_tc_tiling_on_sc=False),
    )(inputs, embed_table)


_NT = 512


def _mm_body(p_ref, w_ref, b_ref, o_ref):
    acc = lax.dot_general(
        p_ref[...], w_ref[...],
        (((1,), (1,)), ((), ())),
        preferred_element_type=jnp.float32,
    )
    o_ref[...] = acc + b_ref[...]


def _matmul(pooled, W, b):
    n_blocks = (_OUT + _NT - 1) // _NT
    n_pad = n_blocks * _NT
    W_p = jnp.pad(W, ((0, n_pad - _OUT), (0, 0)))
    b_p = jnp.pad(b, (0, n_pad - _OUT)).reshape(1, n_pad)
    return pl.pallas_call(
        _mm_body,
        grid=(n_blocks,),
        in_specs=[
            pl.BlockSpec((_B, _D), lambda j: (0, 0)),
            pl.BlockSpec((_NT, _D), lambda j: (j, 0)),
            pl.BlockSpec((1, _NT), lambda j: (0, j)),
        ],
        out_specs=pl.BlockSpec((_B, _NT), lambda j: (0, j)),
        out_shape=jax.ShapeDtypeStruct((_B, _OUT), jnp.float32),
    )(pooled, W_p, b_p)


def kernel(inputs, embed_table, W, b):
    pooled = _sc_pool(inputs.astype(jnp.int32), embed_table)
    return _matmul(pooled, W, b)
